# Initial kernel scaffold; baseline (speedup 1.0000x reference)
import jax
import jax.numpy as jnp
from jax.experimental import pallas as pl


def kernel(node_emb, er_src, er_dst, ee_src, ee_dst, ee_weight, W_attn_w, W_attn_b, w0_w, w0_b, W_self_w, W_self_b, W_neigh_w, W_neigh_b):
    raise NotImplementedError("write your pallas kernel here")


# speedup vs baseline: 3.7361x; 3.7361x over previous
import functools

import jax
import jax.numpy as jnp
from jax import lax
from jax.experimental import pallas as pl
from jax.experimental.pallas import tpu as pltpu
from jax.experimental.pallas import tpu_sc as plsc

D = 128
LANES = 16
KCH = D // LANES


def _mm_body(x_ref, wa_ref, ba_ref, wm_ref, bm_ref, wb_ref, ws_ref, bs_ref,
             a_ref, m_ref, b_ref, s_ref):
    x = x_ref[...]
    f32 = jnp.float32
    a_ref[...] = jnp.dot(x, wa_ref[...], preferred_element_type=f32) + ba_ref[...]
    m_ref[...] = jnp.dot(x, wm_ref[...], preferred_element_type=f32) + bm_ref[...]
    b_ref[...] = jnp.dot(x, wb_ref[...], preferred_element_type=f32)
    s_ref[...] = jnp.dot(x, ws_ref[...], preferred_element_type=f32) + bs_ref[...]


def _precompute(x, wa, ba, wm, bm, wb, ws, bs, block_n):
    n = x.shape[0]
    grid = n // block_n
    blk = lambda i: (i, 0)
    w_spec = pl.BlockSpec((D, D), lambda i: (0, 0))
    b_spec = pl.BlockSpec((1, D), lambda i: (0, 0))
    nd_spec = pl.BlockSpec((block_n, D), blk)
    return pl.pallas_call(
        _mm_body,
        grid=(grid,),
        in_specs=[nd_spec, w_spec, b_spec, w_spec, b_spec, w_spec, w_spec,
                  b_spec],
        out_specs=[nd_spec, nd_spec, nd_spec, nd_spec],
        out_shape=[jax.ShapeDtypeStruct((n, D), jnp.float32)] * 4,
    )(x, wa, ba, wm, bm, wb, ws, bs)


def _make_sc_kernel(n2, e, n_cores, n_sub, chunk):
    workers = n_cores * n_sub
    total_chunks = e // chunk
    n_chunks_w = total_chunks // workers
    rem = total_chunks % workers
    rows_per_tile = n2 // n_sub
    groups = chunk // LANES

    mesh = plsc.VectorSubcoreMesh(core_axis_name="c", subcore_axis_name="s")

    @functools.partial(
        pl.kernel,
        out_type=(
            jax.ShapeDtypeStruct((n_cores, n2, D), jnp.float32),
            jax.ShapeDtypeStruct((workers, n2), jnp.float32),
            jax.ShapeDtypeStruct((workers, n2), jnp.float32),
        ),
        mesh=mesh,
        compiler_params=pltpu.CompilerParams(needs_layout_passes=False),
        scratch_types=[
            pltpu.VMEM_SHARED((n2, D), jnp.float32),
            pltpu.VMEM((n2,), jnp.float32),
            pltpu.VMEM((n2,), jnp.float32),
            pltpu.VMEM((chunk,), jnp.int32),
            pltpu.VMEM((chunk,), jnp.int32),
            pltpu.VMEM((chunk, D), jnp.float32),
            pltpu.VMEM((chunk, D), jnp.float32),
            pltpu.VMEM((chunk, D), jnp.float32),
            pltpu.VMEM((chunk,), jnp.int32),
            pltpu.VMEM((chunk,), jnp.float32),
            pltpu.VMEM((KCH * LANES + LANES,), jnp.float32),
            pltpu.SemaphoreType.DMA,
            pltpu.SemaphoreType.DMA,
            pltpu.SemaphoreType.DMA,
        ],
    )
    def sc_kernel(a_hbm, b_hbm, m_hbm, ersrc_hbm, erdst_hbm, eesrc_hbm,
                  eew_hbm, w0_hbm, z128_hbm, z1_hbm,
                  num_out, den_out, ws_out,
                  num_sh, den_t, ws_t, src_i, dst_i, a_buf, b_buf, m_buf,
                  ee_i, ee_wv, w0_v, sem1, sem2, sem3):
        c = lax.axis_index("c")
        s = lax.axis_index("s")
        wid = c * n_sub + s
        r0 = s * rows_per_tile

        pltpu.sync_copy(z128_hbm.at[pl.ds(r0, rows_per_tile)],
                        num_sh.at[pl.ds(r0, rows_per_tile)])
        pltpu.sync_copy(z1_hbm, den_t)
        pltpu.sync_copy(z1_hbm, ws_t)
        pltpu.sync_copy(w0_hbm, w0_v)
        plsc.subcore_barrier()

        w0s = [w0_v[pl.ds(k * LANES, LANES)] for k in range(KCH)]
        wsum = w0s[0]
        for k in range(1, KCH):
            wsum = wsum + w0s[k]
        wb_vec = w0_v[pl.ds(KCH * LANES, LANES)]
        t0 = jnp.sum(wsum) + wb_vec[0]

        lane_iota = lax.iota(jnp.int32, LANES)

        def chunk_body(ci, _):
            off = (ci * workers + wid) * chunk
            pltpu.sync_copy(ersrc_hbm.at[pl.ds(off, chunk)], src_i)
            pltpu.sync_copy(erdst_hbm.at[pl.ds(off, chunk)], dst_i)
            cp1 = pltpu.async_copy(a_hbm.at[dst_i], a_buf, sem1)
            cp2 = pltpu.async_copy(b_hbm.at[src_i], b_buf, sem2)
            cp3 = pltpu.async_copy(m_hbm.at[dst_i], m_buf, sem3)
            cp1.wait()
            cp2.wait()
            cp3.wait()

            def group_body(g, _):
                e0 = g * LANES
                srcv = src_i[pl.ds(e0, LANES)]
                p16 = jnp.zeros((LANES,), jnp.float32)
                for j in range(LANES):
                    ei = e0 + j
                    acc = jnp.zeros((LANES,), jnp.float32)
                    for k in range(KCH):
                        a = a_buf[ei, pl.ds(k * LANES, LANES)]
                        b = b_buf[ei, pl.ds(k * LANES, LANES)]
                        x2 = (a + b) * 2.0
                        acc = acc + w0s[k] / (jnp.exp(x2) + 1.0)
                    t = t0 - 2.0 * jnp.sum(acc)
                    pv = jnp.exp(jnp.full((LANES,), t, jnp.float32))
                    for k in range(KCH):
                        sl = pl.ds(k * LANES, LANES)
                        m_buf[ei, sl] = pv * m_buf[ei, sl]
                    p16 = jnp.where(lane_iota == j, pv, p16)
                plsc.addupdate_scatter(den_t, [srcv], p16)
                return _

            lax.fori_loop(0, groups, group_body, None, unroll=False)
            pltpu.sync_copy(m_buf, num_sh.at[src_i], add=True)

            pltpu.sync_copy(eesrc_hbm.at[pl.ds(off, chunk)], ee_i)
            pltpu.sync_copy(eew_hbm.at[pl.ds(off, chunk)], ee_wv)
            for g in range(groups):
                idx = ee_i[pl.ds(g * LANES, LANES)]
                wv = ee_wv[pl.ds(g * LANES, LANES)]
                plsc.addupdate_scatter(ws_t, [idx], wv)
            return _

        lax.fori_loop(0, n_chunks_w, chunk_body, None, unroll=False)

        if rem:
            @pl.when(wid < rem)
            def _tail():
                chunk_body(n_chunks_w, None)

        plsc.subcore_barrier()

        pltpu.sync_copy(num_sh.at[pl.ds(r0, rows_per_tile)],
                        num_out.at[c, pl.ds(r0, rows_per_tile)])
        pltpu.sync_copy(den_t, den_out.at[wid])
        pltpu.sync_copy(ws_t, ws_out.at[wid])

    return sc_kernel


def _make_combine(n2, workers, block_n):
    grid = n2 // block_n

    def body(s_ref, m_ref, num_ref, den_ref, ws_ref, o_ref):
        den = jnp.sum(den_ref[...], axis=1, keepdims=True)
        ws = jnp.sum(ws_ref[...], axis=1, keepdims=True)
        num = num_ref[0] + num_ref[1]
        agg = num / (den + 1e-9) + ws * m_ref[...]
        o_ref[...] = jnp.tanh(s_ref[...] + agg)

    nd_spec = pl.BlockSpec((block_n, D), lambda i: (i, 0))
    sc_spec = pl.BlockSpec((block_n, workers), lambda i: (i, 0))
    return pl.pallas_call(
        body,
        grid=(grid,),
        in_specs=[
            nd_spec,
            nd_spec,
            pl.BlockSpec((2, block_n, D), lambda i: (0, i, 0)),
            sc_spec,
            sc_spec,
        ],
        out_specs=nd_spec,
        out_shape=jax.ShapeDtypeStruct((n2, D), jnp.float32),
    )


def kernel(node_emb, er_src, er_dst, ee_src, ee_dst, ee_weight,
           W_attn_w, W_attn_b, w0_w, w0_b, W_self_w, W_self_b,
           W_neigh_w, W_neigh_b):
    n, d = node_emb.shape
    e = er_src.shape[0]
    assert d == D

    block_n = 1024
    n2 = ((n + block_n - 1) // block_n) * block_n
    x2 = jnp.pad(node_emb, ((0, n2 - n), (0, 0)))

    wa = W_attn_w[:, :D].T
    ba = W_attn_b.reshape(1, D)
    wm = W_neigh_w.T
    bm = W_neigh_b.reshape(1, D)
    wb = W_attn_w[:, D:].T
    ws_w = W_self_w.T
    bs = W_self_b.reshape(1, D)
    w0full = jnp.concatenate([w0_w.reshape(D),
                              jnp.full((LANES,), w0_b[0], jnp.float32)])

    amat, mmat, bmat, smat = _precompute(x2, wa, ba, wm, bm, wb, ws_w, bs,
                                         block_n=block_n)

    ersrc32 = er_src.astype(jnp.int32)
    erdst32 = er_dst.astype(jnp.int32)
    eesrc32 = ee_src.astype(jnp.int32)
    z128 = jnp.zeros((n2, D), jnp.float32)
    z1 = jnp.zeros((n2,), jnp.float32)

    sc = _make_sc_kernel(n2, e, n_cores=2, n_sub=16, chunk=64)
    num, den, wsacc = sc(amat, bmat, mmat, ersrc32, erdst32, eesrc32,
                         ee_weight, w0full, z128, z1)

    combine = _make_combine(n2, workers=32, block_n=block_n)
    out2 = combine(smat, mmat, num, den.T, wsacc.T)
    return out2[:n]


# speedup vs baseline: 7.7626x; 2.0777x over previous
import functools

import jax
import jax.numpy as jnp
from jax import lax
from jax.experimental import pallas as pl
from jax.experimental.pallas import tpu as pltpu
from jax.experimental.pallas import tpu_sc as plsc

D = 128
LANES = 16
KCH = D // LANES


def _mm_body(x_ref, wa_ref, ba_ref, wm_ref, bm_ref, wb_ref, ws_ref, bs_ref,
             a_ref, m_ref, b_ref, s_ref):
    x = x_ref[...]
    f32 = jnp.float32
    a_ref[...] = jnp.dot(x, wa_ref[...], preferred_element_type=f32) + ba_ref[...]
    m_ref[...] = jnp.dot(x, wm_ref[...], preferred_element_type=f32) + bm_ref[...]
    b_ref[...] = jnp.dot(x, wb_ref[...], preferred_element_type=f32)
    s_ref[...] = jnp.dot(x, ws_ref[...], preferred_element_type=f32) + bs_ref[...]


def _precompute(x, wa, ba, wm, bm, wb, ws, bs, block_n):
    n = x.shape[0]
    grid = n // block_n
    blk = lambda i: (i, 0)
    w_spec = pl.BlockSpec((D, D), lambda i: (0, 0))
    b_spec = pl.BlockSpec((1, D), lambda i: (0, 0))
    nd_spec = pl.BlockSpec((block_n, D), blk)
    return pl.pallas_call(
        _mm_body,
        grid=(grid,),
        in_specs=[nd_spec, w_spec, b_spec, w_spec, b_spec, w_spec, w_spec,
                  b_spec],
        out_specs=[nd_spec, nd_spec, nd_spec, nd_spec],
        out_shape=[jax.ShapeDtypeStruct((n, D), jnp.float32)] * 4,
    )(x, wa, ba, wm, bm, wb, ws, bs)


def _make_sc_kernel(n2, e, n_cores, n_sub, chunk):
    workers = n_cores * n_sub
    total_chunks = e // chunk
    n_chunks_w = total_chunks // workers
    rem = total_chunks % workers
    rows_per_tile = n2 // n_sub
    groups = chunk // LANES

    mesh = plsc.VectorSubcoreMesh(core_axis_name="c", subcore_axis_name="s")

    @functools.partial(
        pl.kernel,
        out_type=(
            jax.ShapeDtypeStruct((n_cores, n2, D), jnp.float32),
            jax.ShapeDtypeStruct((workers, n2), jnp.float32),
            jax.ShapeDtypeStruct((workers, n2), jnp.float32),
        ),
        mesh=mesh,
        compiler_params=pltpu.CompilerParams(needs_layout_passes=False),
        scratch_types=[
            pltpu.VMEM_SHARED((n2, D), jnp.float32),
            pltpu.VMEM((n2,), jnp.float32),
            pltpu.VMEM((n2,), jnp.float32),
            pltpu.VMEM((chunk,), jnp.int32),
            pltpu.VMEM((chunk,), jnp.int32),
            pltpu.VMEM((chunk, D), jnp.float32),
            pltpu.VMEM((chunk, D), jnp.float32),
            pltpu.VMEM((chunk, D), jnp.float32),
            pltpu.VMEM((chunk,), jnp.int32),
            pltpu.VMEM((chunk,), jnp.float32),
            pltpu.VMEM((KCH * LANES + LANES,), jnp.float32),
            pltpu.VMEM((LANES * 17 + LANES,), jnp.float32),
            pltpu.SemaphoreType.DMA,
            pltpu.SemaphoreType.DMA,
            pltpu.SemaphoreType.DMA,
        ],
    )
    def sc_kernel(a_hbm, b_hbm, m_hbm, ersrc_hbm, erdst_hbm, eesrc_hbm,
                  eew_hbm, w0_hbm, z128_hbm, z1_hbm,
                  num_out, den_out, ws_out,
                  num_sh, den_t, ws_t, src_i, dst_i, a_buf, b_buf, m_buf,
                  ee_i, ee_wv, w0_v, tr_buf, sem1, sem2, sem3):
        c = lax.axis_index("c")
        s = lax.axis_index("s")
        wid = c * n_sub + s
        r0 = s * rows_per_tile

        pltpu.sync_copy(z128_hbm.at[pl.ds(r0, rows_per_tile)],
                        num_sh.at[pl.ds(r0, rows_per_tile)])
        pltpu.sync_copy(z1_hbm, den_t)
        pltpu.sync_copy(z1_hbm, ws_t)
        pltpu.sync_copy(w0_hbm, w0_v)
        plsc.subcore_barrier()

        w0s = [w0_v[pl.ds(k * LANES, LANES)] for k in range(KCH)]
        wsum = w0s[0]
        for k in range(1, KCH):
            wsum = wsum + w0s[k]
        wb_vec = w0_v[pl.ds(KCH * LANES, LANES)]
        t0 = jnp.sum(wsum) + wb_vec[0]
        t0v = jnp.full((LANES,), t0, jnp.float32)

        lane_iota = lax.iota(jnp.int32, LANES)
        iota17 = lane_iota * 17

        def chunk_body(ci, _):
            off = (ci * workers + wid) * chunk
            pltpu.sync_copy(ersrc_hbm.at[pl.ds(off, chunk)], src_i)
            pltpu.sync_copy(erdst_hbm.at[pl.ds(off, chunk)], dst_i)
            cp1 = pltpu.async_copy(a_hbm.at[dst_i], a_buf, sem1)
            cp2 = pltpu.async_copy(b_hbm.at[src_i], b_buf, sem2)
            cp3 = pltpu.async_copy(m_hbm.at[dst_i], m_buf, sem3)
            cp1.wait()
            cp2.wait()
            cp3.wait()

            def group_body(g, _):
                e0 = g * LANES
                srcv = src_i[pl.ds(e0, LANES)]
                for j in range(LANES):
                    ei = e0 + j
                    d = [jnp.exp(a_buf[ei, pl.ds(k * LANES, LANES)]
                                 + b_buf[ei, pl.ds(k * LANES, LANES)]) + 1.0
                         for k in range(KCH)]
                    n2 = [w0s[2 * i] * d[2 * i + 1] + w0s[2 * i + 1] * d[2 * i]
                          for i in range(4)]
                    d2 = [d[2 * i] * d[2 * i + 1] for i in range(4)]
                    n4 = [n2[0] * d2[1] + n2[1] * d2[0],
                          n2[2] * d2[3] + n2[3] * d2[2]]
                    d4 = [d2[0] * d2[1], d2[2] * d2[3]]
                    n8 = n4[0] * d4[1] + n4[1] * d4[0]
                    d8 = d4[0] * d4[1]
                    plsc.store_scatter(tr_buf, [lane_iota + (17 * j)], n8 / d8)
                tsum = plsc.load_gather(tr_buf, [iota17])
                for cidx in range(1, LANES):
                    tsum = tsum + plsc.load_gather(tr_buf, [iota17 + cidx])
                p16 = jnp.exp(t0v - 2.0 * tsum)
                plsc.addupdate_scatter(den_t, [srcv], p16)
                tr_buf[pl.ds(LANES * 17, LANES)] = p16
                for j in range(LANES):
                    ei = e0 + j
                    pvj = plsc.load_gather(
                        tr_buf, [jnp.full((LANES,), LANES * 17 + j, jnp.int32)])
                    for k in range(KCH):
                        sl = pl.ds(k * LANES, LANES)
                        m_buf[ei, sl] = pvj * m_buf[ei, sl]
                return _

            lax.fori_loop(0, groups, group_body, None, unroll=False)
            pltpu.sync_copy(m_buf, num_sh.at[src_i], add=True)

            pltpu.sync_copy(eesrc_hbm.at[pl.ds(off, chunk)], ee_i)
            pltpu.sync_copy(eew_hbm.at[pl.ds(off, chunk)], ee_wv)
            for g in range(groups):
                idx = ee_i[pl.ds(g * LANES, LANES)]
                wv = ee_wv[pl.ds(g * LANES, LANES)]
                plsc.addupdate_scatter(ws_t, [idx], wv)
            return _

        lax.fori_loop(0, n_chunks_w, chunk_body, None, unroll=False)

        if rem:
            @pl.when(wid < rem)
            def _tail():
                chunk_body(n_chunks_w, None)

        plsc.subcore_barrier()

        pltpu.sync_copy(num_sh.at[pl.ds(r0, rows_per_tile)],
                        num_out.at[c, pl.ds(r0, rows_per_tile)])
        pltpu.sync_copy(den_t, den_out.at[wid])
        pltpu.sync_copy(ws_t, ws_out.at[wid])

    return sc_kernel


def _make_combine(n2, workers, block_n):
    grid = n2 // block_n

    def body(s_ref, m_ref, num_ref, den_ref, ws_ref, o_ref):
        den = jnp.sum(den_ref[...], axis=1, keepdims=True)
        ws = jnp.sum(ws_ref[...], axis=1, keepdims=True)
        num = num_ref[0] + num_ref[1]
        agg = num / (den + 1e-9) + ws * m_ref[...]
        o_ref[...] = jnp.tanh(s_ref[...] + agg)

    nd_spec = pl.BlockSpec((block_n, D), lambda i: (i, 0))
    sc_spec = pl.BlockSpec((block_n, workers), lambda i: (i, 0))
    return pl.pallas_call(
        body,
        grid=(grid,),
        in_specs=[
            nd_spec,
            nd_spec,
            pl.BlockSpec((2, block_n, D), lambda i: (0, i, 0)),
            sc_spec,
            sc_spec,
        ],
        out_specs=nd_spec,
        out_shape=jax.ShapeDtypeStruct((n2, D), jnp.float32),
    )


def kernel(node_emb, er_src, er_dst, ee_src, ee_dst, ee_weight,
           W_attn_w, W_attn_b, w0_w, w0_b, W_self_w, W_self_b,
           W_neigh_w, W_neigh_b):
    n, d = node_emb.shape
    e = er_src.shape[0]
    assert d == D

    block_n = 1024
    n2 = ((n + block_n - 1) // block_n) * block_n
    x2 = jnp.pad(node_emb, ((0, n2 - n), (0, 0)))

    wa = W_attn_w[:, :D].T * 2.0
    ba = (W_attn_b * 2.0).reshape(1, D)
    wm = W_neigh_w.T
    bm = W_neigh_b.reshape(1, D)
    wb = W_attn_w[:, D:].T * 2.0
    ws_w = W_self_w.T
    bs = W_self_b.reshape(1, D)
    w0full = jnp.concatenate([w0_w.reshape(D),
                              jnp.full((LANES,), w0_b[0], jnp.float32)])

    amat, mmat, bmat, smat = _precompute(x2, wa, ba, wm, bm, wb, ws_w, bs,
                                         block_n=block_n)

    ersrc32 = er_src.astype(jnp.int32)
    erdst32 = er_dst.astype(jnp.int32)
    eesrc32 = ee_src.astype(jnp.int32)
    z128 = jnp.zeros((n2, D), jnp.float32)
    z1 = jnp.zeros((n2,), jnp.float32)

    sc = _make_sc_kernel(n2, e, n_cores=2, n_sub=16, chunk=64)
    num, den, wsacc = sc(amat, bmat, mmat, ersrc32, erdst32, eesrc32,
                         ee_weight, w0full, z128, z1)

    combine = _make_combine(n2, workers=32, block_n=block_n)
    out2 = combine(smat, mmat, num, den.T, wsacc.T)
    return out2[:n]
